# Initial kernel scaffold; baseline (speedup 1.0000x reference)
import jax
import jax.numpy as jnp
from jax.experimental import pallas as pl


def kernel(x, edge_index, W1, b1, W2, b2):
    raise NotImplementedError("write your pallas kernel here")


# speedup vs baseline: 19.9765x; 19.9765x over previous
import functools

import jax
import jax.numpy as jnp
from jax import lax
from jax.experimental import pallas as pl
from jax.experimental.pallas import tpu as pltpu
from jax.experimental.pallas import tpu_sc as plsc

N = 10000
NPAD = 10240
E = 320000
CH = 128
NW = 32
ROWS_PER_W = 79
EPAD = NW * ROWS_PER_W * CH
EROWS = EPAD // CH
D_HID = 16
NROWS_PER_TILE = NPAD // 16

_mesh = plsc.VectorSubcoreMesh(core_axis_name="c", subcore_axis_name="s")
_sc_params = pltpu.CompilerParams(
    needs_layout_passes=False, use_tc_tiling_on_sc=False
)


@functools.partial(
    pl.kernel,
    mesh=_mesh,
    out_type=jax.ShapeDtypeStruct((NW, NPAD), jnp.int32),
    compiler_params=_sc_params,
    scratch_types=[
        pltpu.VMEM((NPAD,), jnp.int32),
        pltpu.VMEM((CH,), jnp.int32),
    ],
)
def _hist_sc(dstM, out, hist, dbuf):
    c = lax.axis_index("c")
    s = lax.axis_index("s")
    w = s * 2 + c
    zero16 = jnp.zeros((16,), jnp.int32)

    def zbody(i, carry):
        hist[pl.ds(i * 16, 16)] = zero16
        return carry

    lax.fori_loop(0, NPAD // 16, zbody, 0)

    ones16 = jnp.ones((16,), jnp.int32)

    def rbody(i, carry):
        r = w * ROWS_PER_W + i
        pltpu.sync_copy(dstM.at[r], dbuf)
        for k in range(CH // 16):
            d = dbuf[pl.ds(k * 16, 16)]
            plsc.addupdate_scatter(hist, [d], ones16)
        return carry

    lax.fori_loop(0, ROWS_PER_W, rbody, 0)
    pltpu.sync_copy(hist, out.at[w])


@functools.partial(
    pl.kernel,
    mesh=_mesh,
    out_type=jax.ShapeDtypeStruct((2, NPAD, D_HID), jnp.float32),
    compiler_params=_sc_params,
    scratch_types=[
        pltpu.VMEM_SHARED((NPAD, D_HID), jnp.float32),
        pltpu.VMEM((CH,), jnp.int32),
        pltpu.VMEM((1, CH), jnp.int32),
        pltpu.VMEM((CH, D_HID), jnp.float32),
        pltpu.SemaphoreType.DMA,
    ],
)
def _prop_sc(g, srcM, dstM, zeros, out, acc, sbuf, dbuf, rows, sem):
    c = lax.axis_index("c")
    s = lax.axis_index("s")
    w = s * 2 + c
    pltpu.sync_copy(
        zeros.at[pl.ds(s * NROWS_PER_TILE, NROWS_PER_TILE)],
        acc.at[pl.ds(s * NROWS_PER_TILE, NROWS_PER_TILE)],
    )
    plsc.subcore_barrier()

    def rbody(i, carry):
        r = w * ROWS_PER_W + i
        pltpu.sync_copy(srcM.at[r], sbuf)
        pltpu.sync_copy(dstM.at[pl.ds(r, 1)], dbuf)
        pltpu.async_copy(g.at[sbuf], rows, sem).wait()
        pltpu.sync_copy(rows, acc.at[dbuf.at[0]], add=True)
        return carry

    lax.fori_loop(0, ROWS_PER_W, rbody, 0)
    plsc.subcore_barrier()
    pltpu.sync_copy(
        acc.at[pl.ds(s * NROWS_PER_TILE, NROWS_PER_TILE)],
        out.at[c, pl.ds(s * NROWS_PER_TILE, NROWS_PER_TILE)],
    )


def _mm1_body(x_ref, w_ref, hist_ref, g_ref, dis_ref):
    h = jnp.dot(x_ref[...], w_ref[...], preferred_element_type=jnp.float32)
    deg = jnp.sum(hist_ref[...], axis=1, keepdims=True).astype(jnp.float32) + 1.0
    dis = lax.rsqrt(deg)
    dis_ref[...] = dis
    g_ref[...] = h * dis


def _act_body(a_ref, g_ref, dis_ref, b_ref, o_ref):
    dis = dis_ref[...]
    acc = a_ref[0] + a_ref[1] + g_ref[...]
    h = jnp.maximum(acc * dis + b_ref[...], 0.0)
    o_ref[...] = h * dis


def _mm2_body(a_ref, g_ref, dis_ref, w_ref, b_ref, o_ref):
    p = (a_ref[0] + a_ref[1] + g_ref[...]) * dis_ref[...]
    o_ref[...] = (
        jnp.dot(p, w_ref[...], preferred_element_type=jnp.float32) + b_ref[...]
    )


_BM = 256
_GRID = NPAD // _BM


def _mm1(x_pad, W1, hist_t):
    return pl.pallas_call(
        _mm1_body,
        grid=(_GRID,),
        in_specs=[
            pl.BlockSpec((_BM, 128), lambda i: (i, 0)),
            pl.BlockSpec((128, D_HID), lambda i: (0, 0)),
            pl.BlockSpec((_BM, NW), lambda i: (i, 0)),
        ],
        out_specs=[
            pl.BlockSpec((_BM, D_HID), lambda i: (i, 0)),
            pl.BlockSpec((_BM, 1), lambda i: (i, 0)),
        ],
        out_shape=[
            jax.ShapeDtypeStruct((NPAD, D_HID), jnp.float32),
            jax.ShapeDtypeStruct((NPAD, 1), jnp.float32),
        ],
    )(x_pad, W1, hist_t)


def _act(acc, g1, dis, b1):
    return pl.pallas_call(
        _act_body,
        grid=(_GRID,),
        in_specs=[
            pl.BlockSpec((2, _BM, D_HID), lambda i: (0, i, 0)),
            pl.BlockSpec((_BM, D_HID), lambda i: (i, 0)),
            pl.BlockSpec((_BM, 1), lambda i: (i, 0)),
            pl.BlockSpec((1, D_HID), lambda i: (0, 0)),
        ],
        out_specs=pl.BlockSpec((_BM, D_HID), lambda i: (i, 0)),
        out_shape=jax.ShapeDtypeStruct((NPAD, D_HID), jnp.float32),
    )(acc, g1, dis, b1)


def _mm2(acc, g2, dis, W2, b2):
    return pl.pallas_call(
        _mm2_body,
        grid=(_GRID,),
        in_specs=[
            pl.BlockSpec((2, _BM, D_HID), lambda i: (0, i, 0)),
            pl.BlockSpec((_BM, D_HID), lambda i: (i, 0)),
            pl.BlockSpec((_BM, 1), lambda i: (i, 0)),
            pl.BlockSpec((D_HID, 128), lambda i: (0, 0)),
            pl.BlockSpec((1, 128), lambda i: (0, 0)),
        ],
        out_specs=pl.BlockSpec((_BM, 128), lambda i: (i, 0)),
        out_shape=jax.ShapeDtypeStruct((NPAD, 128), jnp.float32),
    )(acc, g2, dis, W2, b2)


def kernel(x, edge_index, W1, b1, W2, b2):
    ei = edge_index.astype(jnp.int32)
    src = ei[0]
    dst = ei[1]
    pad = EPAD - E
    src_p = jnp.concatenate([src, jnp.zeros((pad,), jnp.int32)])
    dst_p = jnp.concatenate([dst, jnp.full((pad,), N, jnp.int32)])
    srcM = src_p.reshape(EROWS, CH)
    dstM = dst_p.reshape(EROWS, CH)

    x_pad = jnp.pad(x, ((0, NPAD - N), (0, 0)))
    zeros = jnp.zeros((NPAD, D_HID), jnp.float32)

    hist = _hist_sc(dstM)
    hist_t = hist.reshape(NW, NPAD).T

    g1, dis = _mm1(x_pad, W1, hist_t)
    acc1 = _prop_sc(g1, srcM, dstM, zeros)
    g2 = _act(acc1, g1, dis, b1.reshape(1, D_HID))
    acc2 = _prop_sc(g2, srcM, dstM, zeros)
    out = _mm2(acc2, g2, dis, W2, b2.reshape(1, 128))
    return out[:N]


# speedup vs baseline: 29.0850x; 1.4560x over previous
import functools

import jax
import jax.numpy as jnp
from jax import lax
from jax.experimental import pallas as pl
from jax.experimental.pallas import tpu as pltpu
from jax.experimental.pallas import tpu_sc as plsc

N = 10000
NPAD = 10240
E = 320000
CH = 128
NW = 32
ROWS_PER_W = 80
EPAD = NW * ROWS_PER_W * CH
EROWS = EPAD // CH
SB = 16
SUPERS = ROWS_PER_W // SB
D_HID = 16
NROWS_PER_TILE = NPAD // 16

_mesh = plsc.VectorSubcoreMesh(core_axis_name="c", subcore_axis_name="s")
_sc_params = pltpu.CompilerParams(
    needs_layout_passes=False, use_tc_tiling_on_sc=False
)


@functools.partial(
    pl.kernel,
    mesh=_mesh,
    out_type=jax.ShapeDtypeStruct((NW, NPAD), jnp.int32),
    compiler_params=_sc_params,
    scratch_types=[
        pltpu.VMEM((NPAD,), jnp.int32),
        pltpu.VMEM((SB, CH), jnp.int32),
    ],
)
def _hist_sc(dstM, out, hist, dbuf):
    c = lax.axis_index("c")
    s = lax.axis_index("s")
    w = s * 2 + c
    zero16 = jnp.zeros((16,), jnp.int32)

    def zbody(i, carry):
        hist[pl.ds(i * 16, 16)] = zero16
        return carry

    lax.fori_loop(0, NPAD // 16, zbody, 0)

    ones16 = jnp.ones((16,), jnp.int32)

    def rbody(i, carry):
        r0 = w * ROWS_PER_W + i * SB
        pltpu.sync_copy(dstM.at[pl.ds(r0, SB)], dbuf)

        def jbody(j, carry2):
            row = dbuf.at[j]
            for k in range(CH // 16):
                d = row[pl.ds(k * 16, 16)]
                plsc.addupdate_scatter(hist, [d], ones16)
            return carry2

        lax.fori_loop(0, SB, jbody, 0)
        return carry

    lax.fori_loop(0, SUPERS, rbody, 0)
    pltpu.sync_copy(hist, out.at[w])


@functools.partial(
    pl.kernel,
    mesh=_mesh,
    out_type=jax.ShapeDtypeStruct((2, NPAD, D_HID), jnp.float32),
    compiler_params=_sc_params,
    scratch_types=[
        pltpu.VMEM_SHARED((NPAD, D_HID), jnp.float32),
        pltpu.VMEM((SB, CH), jnp.int32),
        pltpu.VMEM((SB, CH), jnp.int32),
        pltpu.VMEM((SB, CH, D_HID), jnp.float32),
        pltpu.SemaphoreType.DMA,
        pltpu.SemaphoreType.DMA,
    ],
)
def _prop_sc(g, srcM, dstM, zeros, out, acc, sbuf, dbuf, rows, gsem, ssem):
    c = lax.axis_index("c")
    s = lax.axis_index("s")
    w = s * 2 + c
    pltpu.sync_copy(
        zeros.at[pl.ds(s * NROWS_PER_TILE, NROWS_PER_TILE)],
        acc.at[pl.ds(s * NROWS_PER_TILE, NROWS_PER_TILE)],
    )
    plsc.subcore_barrier()

    def rbody(i, carry):
        r0 = w * ROWS_PER_W + i * SB
        pltpu.sync_copy(srcM.at[pl.ds(r0, SB)], sbuf)
        pltpu.sync_copy(dstM.at[pl.ds(r0, SB)], dbuf)
        gds = [
            pltpu.async_copy(g.at[sbuf.at[j]], rows.at[j], gsem)
            for j in range(SB)
        ]
        for d in gds:
            d.wait()
        sds = [
            pltpu.async_copy(rows.at[j], acc.at[dbuf.at[j]], ssem, add=True)
            for j in range(SB)
        ]
        for d in sds:
            d.wait()
        return carry

    lax.fori_loop(0, SUPERS, rbody, 0)
    plsc.subcore_barrier()
    pltpu.sync_copy(
        acc.at[pl.ds(s * NROWS_PER_TILE, NROWS_PER_TILE)],
        out.at[c, pl.ds(s * NROWS_PER_TILE, NROWS_PER_TILE)],
    )


def _mm1_body(x_ref, w_ref, hist_ref, g_ref, dis_ref):
    h = jnp.dot(x_ref[...], w_ref[...], preferred_element_type=jnp.float32)
    deg = jnp.sum(hist_ref[...], axis=1, keepdims=True).astype(jnp.float32) + 1.0
    dis = lax.rsqrt(deg)
    dis_ref[...] = dis
    g_ref[...] = h * dis


def _act_body(a_ref, g_ref, dis_ref, b_ref, o_ref):
    dis = dis_ref[...]
    acc = a_ref[0] + a_ref[1] + g_ref[...]
    h = jnp.maximum(acc * dis + b_ref[...], 0.0)
    o_ref[...] = h * dis


def _mm2_body(a_ref, g_ref, dis_ref, w_ref, b_ref, o_ref):
    p = (a_ref[0] + a_ref[1] + g_ref[...]) * dis_ref[...]
    o_ref[...] = (
        jnp.dot(p, w_ref[...], preferred_element_type=jnp.float32) + b_ref[...]
    )


_BM = 256
_GRID = NPAD // _BM


def _mm1(x_pad, W1, hist_t):
    return pl.pallas_call(
        _mm1_body,
        grid=(_GRID,),
        in_specs=[
            pl.BlockSpec((_BM, 128), lambda i: (i, 0)),
            pl.BlockSpec((128, D_HID), lambda i: (0, 0)),
            pl.BlockSpec((_BM, NW), lambda i: (i, 0)),
        ],
        out_specs=[
            pl.BlockSpec((_BM, D_HID), lambda i: (i, 0)),
            pl.BlockSpec((_BM, 1), lambda i: (i, 0)),
        ],
        out_shape=[
            jax.ShapeDtypeStruct((NPAD, D_HID), jnp.float32),
            jax.ShapeDtypeStruct((NPAD, 1), jnp.float32),
        ],
    )(x_pad, W1, hist_t)


def _act(acc, g1, dis, b1):
    return pl.pallas_call(
        _act_body,
        grid=(_GRID,),
        in_specs=[
            pl.BlockSpec((2, _BM, D_HID), lambda i: (0, i, 0)),
            pl.BlockSpec((_BM, D_HID), lambda i: (i, 0)),
            pl.BlockSpec((_BM, 1), lambda i: (i, 0)),
            pl.BlockSpec((1, D_HID), lambda i: (0, 0)),
        ],
        out_specs=pl.BlockSpec((_BM, D_HID), lambda i: (i, 0)),
        out_shape=jax.ShapeDtypeStruct((NPAD, D_HID), jnp.float32),
    )(acc, g1, dis, b1)


def _mm2(acc, g2, dis, W2, b2):
    return pl.pallas_call(
        _mm2_body,
        grid=(_GRID,),
        in_specs=[
            pl.BlockSpec((2, _BM, D_HID), lambda i: (0, i, 0)),
            pl.BlockSpec((_BM, D_HID), lambda i: (i, 0)),
            pl.BlockSpec((_BM, 1), lambda i: (i, 0)),
            pl.BlockSpec((D_HID, 128), lambda i: (0, 0)),
            pl.BlockSpec((1, 128), lambda i: (0, 0)),
        ],
        out_specs=pl.BlockSpec((_BM, 128), lambda i: (i, 0)),
        out_shape=jax.ShapeDtypeStruct((NPAD, 128), jnp.float32),
    )(acc, g2, dis, W2, b2)


def kernel(x, edge_index, W1, b1, W2, b2):
    ei = edge_index.astype(jnp.int32)
    src = ei[0]
    dst = ei[1]
    pad = EPAD - E
    src_p = jnp.concatenate([src, jnp.zeros((pad,), jnp.int32)])
    dst_p = jnp.concatenate([dst, jnp.full((pad,), N, jnp.int32)])
    srcM = src_p.reshape(EROWS, CH)
    dstM = dst_p.reshape(EROWS, CH)

    x_pad = jnp.pad(x, ((0, NPAD - N), (0, 0)))
    zeros = jnp.zeros((NPAD, D_HID), jnp.float32)

    hist = _hist_sc(dstM)
    hist_t = hist.reshape(NW, NPAD).T

    g1, dis = _mm1(x_pad, W1, hist_t)
    acc1 = _prop_sc(g1, srcM, dstM, zeros)
    g2 = _act(acc1, g1, dis, b1.reshape(1, D_HID))
    acc2 = _prop_sc(g2, srcM, dstM, zeros)
    out = _mm2(acc2, g2, dis, W2, b2.reshape(1, 128))
    return out[:N]


# speedup vs baseline: 42.9092x; 1.4753x over previous
import functools

import jax
import jax.numpy as jnp
from jax import lax
from jax.experimental import pallas as pl
from jax.experimental.pallas import tpu as pltpu
from jax.experimental.pallas import tpu_sc as plsc

N = 10000
NPAD = 10240
E = 320000
CH = 128
NW = 32
ROWS_PER_W = 80
EPAD = NW * ROWS_PER_W * CH
EROWS = EPAD // CH
SB = 16
SUPERS = ROWS_PER_W // SB
D_HID = 16
NROWS_PER_TILE = NPAD // 16

_mesh = plsc.VectorSubcoreMesh(core_axis_name="c", subcore_axis_name="s")
_sc_params = pltpu.CompilerParams(
    needs_layout_passes=False, use_tc_tiling_on_sc=False
)


@functools.partial(
    pl.kernel,
    mesh=_mesh,
    out_type=jax.ShapeDtypeStruct((NW, NPAD), jnp.int32),
    compiler_params=_sc_params,
    scratch_types=[
        pltpu.VMEM((NPAD,), jnp.int32),
        pltpu.VMEM((SB, CH), jnp.int32),
    ],
)
def _hist_sc(dstM, out, hist, dbuf):
    c = lax.axis_index("c")
    s = lax.axis_index("s")
    w = s * 2 + c
    zero16 = jnp.zeros((16,), jnp.int32)

    def zbody(i, carry):
        hist[pl.ds(i * 16, 16)] = zero16
        return carry

    lax.fori_loop(0, NPAD // 16, zbody, 0)

    ones16 = jnp.ones((16,), jnp.int32)

    def rbody(i, carry):
        r0 = w * ROWS_PER_W + i * SB
        pltpu.sync_copy(dstM.at[pl.ds(r0, SB)], dbuf)

        def jbody(j, carry2):
            row = dbuf.at[j]
            for k in range(CH // 16):
                d = row[pl.ds(k * 16, 16)]
                plsc.addupdate_scatter(hist, [d], ones16)
            return carry2

        lax.fori_loop(0, SB, jbody, 0)
        return carry

    lax.fori_loop(0, SUPERS, rbody, 0)
    pltpu.sync_copy(hist, out.at[w])


@functools.partial(
    pl.kernel,
    mesh=_mesh,
    out_type=jax.ShapeDtypeStruct((2, NPAD, D_HID), jnp.float32),
    compiler_params=_sc_params,
    scratch_types=[
        pltpu.VMEM_SHARED((NPAD, D_HID), jnp.float32),
        pltpu.VMEM_SHARED((NPAD, D_HID), jnp.float32),
        pltpu.VMEM((SB, CH), jnp.int32),
        pltpu.VMEM((SB, CH), jnp.int32),
        pltpu.VMEM((SB, CH, D_HID), jnp.float32),
        pltpu.SemaphoreType.DMA,
        pltpu.SemaphoreType.DMA,
    ],
)
def _prop_sc(g, srcM, dstM, zeros, out, acc, gtab, sbuf, dbuf, rows, gsem, ssem):
    c = lax.axis_index("c")
    s = lax.axis_index("s")
    w = s * 2 + c
    pltpu.sync_copy(
        g.at[pl.ds(s * NROWS_PER_TILE, NROWS_PER_TILE)],
        gtab.at[pl.ds(s * NROWS_PER_TILE, NROWS_PER_TILE)],
    )
    pltpu.sync_copy(
        zeros.at[pl.ds(s * NROWS_PER_TILE, NROWS_PER_TILE)],
        acc.at[pl.ds(s * NROWS_PER_TILE, NROWS_PER_TILE)],
    )
    plsc.subcore_barrier()

    def rbody(i, carry):
        r0 = w * ROWS_PER_W + i * SB
        pltpu.sync_copy(srcM.at[pl.ds(r0, SB)], sbuf)
        pltpu.sync_copy(dstM.at[pl.ds(r0, SB)], dbuf)
        gds = [
            pltpu.async_copy(gtab.at[sbuf.at[j]], rows.at[j], gsem)
            for j in range(SB)
        ]
        for d in gds:
            d.wait()
        sds = [
            pltpu.async_copy(rows.at[j], acc.at[dbuf.at[j]], ssem, add=True)
            for j in range(SB)
        ]
        for d in sds:
            d.wait()
        return carry

    lax.fori_loop(0, SUPERS, rbody, 0)
    plsc.subcore_barrier()
    pltpu.sync_copy(
        acc.at[pl.ds(s * NROWS_PER_TILE, NROWS_PER_TILE)],
        out.at[c, pl.ds(s * NROWS_PER_TILE, NROWS_PER_TILE)],
    )


def _mm1_body(x_ref, w_ref, hist_ref, g_ref, dis_ref):
    h = jnp.dot(x_ref[...], w_ref[...], preferred_element_type=jnp.float32)
    deg = jnp.sum(hist_ref[...], axis=1, keepdims=True).astype(jnp.float32) + 1.0
    dis = lax.rsqrt(deg)
    dis_ref[...] = dis
    g_ref[...] = h * dis


def _act_body(a_ref, g_ref, dis_ref, b_ref, o_ref):
    dis = dis_ref[...]
    acc = a_ref[0] + a_ref[1] + g_ref[...]
    h = jnp.maximum(acc * dis + b_ref[...], 0.0)
    o_ref[...] = h * dis


def _mm2_body(a_ref, g_ref, dis_ref, w_ref, b_ref, o_ref):
    p = (a_ref[0] + a_ref[1] + g_ref[...]) * dis_ref[...]
    o_ref[...] = (
        jnp.dot(p, w_ref[...], preferred_element_type=jnp.float32) + b_ref[...]
    )


_BM = 256
_GRID = NPAD // _BM


def _mm1(x_pad, W1, hist_t):
    return pl.pallas_call(
        _mm1_body,
        grid=(_GRID,),
        in_specs=[
            pl.BlockSpec((_BM, 128), lambda i: (i, 0)),
            pl.BlockSpec((128, D_HID), lambda i: (0, 0)),
            pl.BlockSpec((_BM, NW), lambda i: (i, 0)),
        ],
        out_specs=[
            pl.BlockSpec((_BM, D_HID), lambda i: (i, 0)),
            pl.BlockSpec((_BM, 1), lambda i: (i, 0)),
        ],
        out_shape=[
            jax.ShapeDtypeStruct((NPAD, D_HID), jnp.float32),
            jax.ShapeDtypeStruct((NPAD, 1), jnp.float32),
        ],
    )(x_pad, W1, hist_t)


def _act(acc, g1, dis, b1):
    return pl.pallas_call(
        _act_body,
        grid=(_GRID,),
        in_specs=[
            pl.BlockSpec((2, _BM, D_HID), lambda i: (0, i, 0)),
            pl.BlockSpec((_BM, D_HID), lambda i: (i, 0)),
            pl.BlockSpec((_BM, 1), lambda i: (i, 0)),
            pl.BlockSpec((1, D_HID), lambda i: (0, 0)),
        ],
        out_specs=pl.BlockSpec((_BM, D_HID), lambda i: (i, 0)),
        out_shape=jax.ShapeDtypeStruct((NPAD, D_HID), jnp.float32),
    )(acc, g1, dis, b1)


def _mm2(acc, g2, dis, W2, b2):
    return pl.pallas_call(
        _mm2_body,
        grid=(_GRID,),
        in_specs=[
            pl.BlockSpec((2, _BM, D_HID), lambda i: (0, i, 0)),
            pl.BlockSpec((_BM, D_HID), lambda i: (i, 0)),
            pl.BlockSpec((_BM, 1), lambda i: (i, 0)),
            pl.BlockSpec((D_HID, 128), lambda i: (0, 0)),
            pl.BlockSpec((1, 128), lambda i: (0, 0)),
        ],
        out_specs=pl.BlockSpec((_BM, 128), lambda i: (i, 0)),
        out_shape=jax.ShapeDtypeStruct((NPAD, 128), jnp.float32),
    )(acc, g2, dis, W2, b2)


def kernel(x, edge_index, W1, b1, W2, b2):
    ei = edge_index.astype(jnp.int32)
    src = ei[0]
    dst = ei[1]
    pad = EPAD - E
    src_p = jnp.concatenate([src, jnp.zeros((pad,), jnp.int32)])
    dst_p = jnp.concatenate([dst, jnp.full((pad,), N, jnp.int32)])
    srcM = src_p.reshape(EROWS, CH)
    dstM = dst_p.reshape(EROWS, CH)

    x_pad = jnp.pad(x, ((0, NPAD - N), (0, 0)))
    zeros = jnp.zeros((NPAD, D_HID), jnp.float32)

    hist = _hist_sc(dstM)
    hist_t = hist.reshape(NW, NPAD).T

    g1, dis = _mm1(x_pad, W1, hist_t)
    acc1 = _prop_sc(g1, srcM, dstM, zeros)
    g2 = _act(acc1, g1, dis, b1.reshape(1, D_HID))
    acc2 = _prop_sc(g2, srcM, dstM, zeros)
    out = _mm2(acc2, g2, dis, W2, b2.reshape(1, 128))
    return out[:N]


# speedup vs baseline: 68.4447x; 1.5951x over previous
import functools

import jax
import jax.numpy as jnp
from jax import lax
from jax.experimental import pallas as pl
from jax.experimental.pallas import tpu as pltpu
from jax.experimental.pallas import tpu_sc as plsc

N = 10000
E = 320000
CH = 128
EROWS = E // CH
NW = 32
ROWS_PER_W = 78
SB = 13
SUPERS = ROWS_PER_W // SB
D_HID = 16
NPT = N // 16
NF = N // 8

_mesh = plsc.VectorSubcoreMesh(core_axis_name="c", subcore_axis_name="s")
_sc_params = pltpu.CompilerParams(
    needs_layout_passes=False, use_tc_tiling_on_sc=False
)


@functools.partial(
    pl.kernel,
    mesh=_mesh,
    out_type=jax.ShapeDtypeStruct((NW, N), jnp.int32),
    compiler_params=_sc_params,
    scratch_types=[
        pltpu.VMEM((N,), jnp.int32),
        pltpu.VMEM((SB, CH), jnp.int32),
    ],
)
def _hist_sc(dstM, out, hist, dbuf):
    c = lax.axis_index("c")
    s = lax.axis_index("s")
    w = s * 2 + c
    zero16 = jnp.zeros((16,), jnp.int32)

    def zbody(i, carry):
        hist[pl.ds(i * 16, 16)] = zero16
        return carry

    lax.fori_loop(0, N // 16, zbody, 0)

    ones16 = jnp.ones((16,), jnp.int32)

    def count_row(row):
        for k in range(CH // 16):
            d = row[pl.ds(k * 16, 16)]
            plsc.addupdate_scatter(hist, [d], ones16)

    def rbody(i, carry):
        r0 = w * ROWS_PER_W + i * SB
        pltpu.sync_copy(dstM.at[pl.ds(r0, SB)], dbuf)

        def jbody(j, carry2):
            count_row(dbuf.at[j])
            return carry2

        lax.fori_loop(0, SB, jbody, 0)
        return carry

    lax.fori_loop(0, SUPERS, rbody, 0)

    @pl.when(w < EROWS - NW * ROWS_PER_W)
    def _tail():
        pltpu.sync_copy(dstM.at[pl.ds(NW * ROWS_PER_W + w, 1)], dbuf.at[pl.ds(0, 1)])
        count_row(dbuf.at[0])

    pltpu.sync_copy(hist, out.at[w])


@functools.partial(
    pl.kernel,
    mesh=_mesh,
    out_type=jax.ShapeDtypeStruct((2, N, D_HID), jnp.float32),
    compiler_params=_sc_params,
    scratch_types=[
        pltpu.VMEM_SHARED((N, D_HID), jnp.float32),
        pltpu.VMEM_SHARED((N, D_HID), jnp.float32),
        pltpu.VMEM((SB, CH), jnp.int32),
        pltpu.VMEM((SB, CH), jnp.int32),
        pltpu.VMEM((SB, CH, D_HID), jnp.float32),
        pltpu.SemaphoreType.DMA,
        pltpu.SemaphoreType.DMA,
    ],
)
def _prop_sc(g, srcM, dstM, zeros, out, acc, gtab, sbuf, dbuf, rows, gsem, ssem):
    c = lax.axis_index("c")
    s = lax.axis_index("s")
    w = s * 2 + c
    pltpu.sync_copy(g.at[pl.ds(s * NPT, NPT)], gtab.at[pl.ds(s * NPT, NPT)])
    pltpu.sync_copy(zeros.at[pl.ds(s * NPT, NPT)], acc.at[pl.ds(s * NPT, NPT)])
    plsc.subcore_barrier()

    def rbody(i, carry):
        r0 = w * ROWS_PER_W + i * SB
        pltpu.sync_copy(srcM.at[pl.ds(r0, SB)], sbuf)
        pltpu.sync_copy(dstM.at[pl.ds(r0, SB)], dbuf)
        gds = [
            pltpu.async_copy(gtab.at[sbuf.at[j]], rows.at[j], gsem)
            for j in range(SB)
        ]
        for d in gds:
            d.wait()
        sds = [
            pltpu.async_copy(rows.at[j], acc.at[dbuf.at[j]], ssem, add=True)
            for j in range(SB)
        ]
        for d in sds:
            d.wait()
        return carry

    lax.fori_loop(0, SUPERS, rbody, 0)

    @pl.when(w < EROWS - NW * ROWS_PER_W)
    def _tail():
        r = NW * ROWS_PER_W + w
        pltpu.sync_copy(srcM.at[pl.ds(r, 1)], sbuf.at[pl.ds(0, 1)])
        pltpu.sync_copy(dstM.at[pl.ds(r, 1)], dbuf.at[pl.ds(0, 1)])
        pltpu.async_copy(gtab.at[sbuf.at[0]], rows.at[0], gsem).wait()
        pltpu.async_copy(rows.at[0], acc.at[dbuf.at[0]], ssem, add=True).wait()

    plsc.subcore_barrier()
    pltpu.sync_copy(
        acc.at[pl.ds(s * NPT, NPT)],
        out.at[c, pl.ds(s * NPT, NPT)],
    )


def _dis_body(hist_ref, dis_ref):
    deg = jnp.sum(hist_ref[...], axis=0, keepdims=True).astype(jnp.float32) + 1.0
    dis_ref[...] = lax.rsqrt(deg)


def _mm1_body(x_ref, w_ref, d8_ref, sel_ref, g_ref, dis_ref):
    dis = jnp.dot(d8_ref[...], sel_ref[...], preferred_element_type=jnp.float32,
                  precision=lax.Precision.HIGHEST)
    dis_ref[...] = dis
    h = jnp.dot(x_ref[...], w_ref[...], preferred_element_type=jnp.float32,
                precision=lax.Precision.HIGHEST)
    g_ref[...] = h * dis


def _act_body(a_ref, g_ref, dis_ref, b_ref, o_ref):
    dis = dis_ref[...]
    acc = a_ref[0] + a_ref[1] + g_ref[...]
    o_ref[...] = jnp.maximum(acc * dis + b_ref[...], 0.0) * dis


def _mm2_body(a_ref, g_ref, dis_ref, w_ref, b_ref, o_ref):
    p = (a_ref[0] + a_ref[1] + g_ref[...]) * dis_ref[...]
    o_ref[...] = (
        jnp.dot(p, w_ref[...], preferred_element_type=jnp.float32,
                precision=lax.Precision.HIGHEST)
        + b_ref[...]
    )


def _dis_row(hist):
    return pl.pallas_call(
        _dis_body,
        out_shape=jax.ShapeDtypeStruct((1, N), jnp.float32),
    )(hist)


def _mm1(x_r, W1_big, d8, sel):
    return pl.pallas_call(
        _mm1_body,
        out_shape=[
            jax.ShapeDtypeStruct((NF, 128), jnp.float32),
            jax.ShapeDtypeStruct((NF, 128), jnp.float32),
        ],
    )(x_r, W1_big, d8, sel)


def _act(acc, g1, dis, b1_flat):
    return pl.pallas_call(
        _act_body,
        out_shape=jax.ShapeDtypeStruct((NF, 128), jnp.float32),
    )(acc, g1, dis, b1_flat)


def _mm2(acc, g2, dis, W2_big, b2_big):
    return pl.pallas_call(
        _mm2_body,
        out_shape=jax.ShapeDtypeStruct((NF, 1024), jnp.float32),
    )(acc, g2, dis, W2_big, b2_big)


def kernel(x, edge_index, W1, b1, W2, b2):
    ei = edge_index.astype(jnp.int32)
    srcM = ei[0].reshape(EROWS, CH)
    dstM = ei[1].reshape(EROWS, CH)

    x_r = x.reshape(NF, 1024)
    eye8 = jnp.eye(8, dtype=jnp.float32)
    W1_big = jnp.kron(eye8, W1)
    W2_big = jnp.kron(eye8, W2)
    b1_flat = jnp.tile(b1, 8).reshape(1, 128)
    b2_big = jnp.tile(b2, 8).reshape(1, 1024)
    zeros = jnp.zeros((N, D_HID), jnp.float32)

    sel = jnp.kron(eye8, jnp.ones((1, 16), jnp.float32))
    hist = _hist_sc(dstM)
    d8 = _dis_row(hist).reshape(NF, 8)

    g1, dis = _mm1(x_r, W1_big, d8, sel)
    acc1 = _prop_sc(g1.reshape(N, D_HID), srcM, dstM, zeros)
    g2 = _act(acc1.reshape(2, NF, 128), g1, dis, b1_flat)
    acc2 = _prop_sc(g2.reshape(N, D_HID), srcM, dstM, zeros)
    out = _mm2(acc2.reshape(2, NF, 128), g2, dis, W2_big, b2_big)
    return out.reshape(N, 128)
